# baseline (speedup 1.0000x reference)
import jax
import jax.numpy as jnp
from jax.experimental import pallas as pl

B, N, S, NS, DP = 4, 8192, 1024, 64, 128


def _safe_unit(v):
    l = jnp.linalg.norm(v, axis=-1, keepdims=True)
    d = jnp.where(l > 0, l, 1.0)
    u = jnp.where(l > 0, v / d, 0.0)
    return u, l


def _index_points(points, idx):
    b = points.shape[0]
    flat = idx.reshape(b, -1)
    out = jnp.take_along_axis(points, flat[:, :, None], axis=1)
    return out.reshape(idx.shape + (points.shape[-1],))


def _fps(xyz, npoint):
    b, n, _ = xyz.shape
    def body(i, state):
        idxs, dists, far = state
        idxs = idxs.at[:, i].set(far)
        centroid = jnp.take_along_axis(xyz, far[:, None, None], axis=1)
        d = jnp.sum((xyz - centroid) ** 2, -1)
        dists = jnp.minimum(dists, d)
        far = jnp.argmax(dists, -1).astype(jnp.int32)
        return idxs, dists, far
    idxs = jnp.zeros((b, npoint), dtype=jnp.int32)
    dists = jnp.full((b, n), 1e10, dtype=xyz.dtype)
    far = jnp.zeros((b,), dtype=jnp.int32)
    idxs, _, _ = jax.lax.fori_loop(0, npoint, body, (idxs, dists, far))
    return idxs


def _bn_relu(x, g, b):
    m = jnp.mean(x, axis=(0, 1, 2), keepdims=True)
    v = jnp.var(x, axis=(0, 1, 2), keepdims=True)
    return jax.nn.relu((x - m) / jnp.sqrt(v + 1e-5) * g + b)


def _maxpool_kernel(h_ref, o_ref):
    o_ref[...] = jnp.max(h_ref[...], axis=1)


def _maxpool(h):
    R, K, C = h.shape
    BLK = 128
    return pl.pallas_call(
        _maxpool_kernel,
        grid=(R // BLK,),
        in_specs=[pl.BlockSpec((BLK, K, C), lambda i: (i, 0, 0))],
        out_specs=pl.BlockSpec((BLK, C), lambda i: (i, 0)),
        out_shape=jax.ShapeDtypeStruct((R, C), h.dtype),
    )(h)


def kernel(xyz, norm, points, pW0, pb0, pg0, pbt0, pW1, pb1, pg1, pbt1,
           mW0, mb0, mg0, mbt0, mW1, mb1, mg1, mbt1):
    eps = 1e-7
    b = xyz.shape[0]
    fps_idx = _fps(xyz, S)
    new_xyz = _index_points(xyz, fps_idx)
    new_norm = _index_points(norm, fps_idx)
    sq = -2.0 * jnp.matmul(new_xyz, xyz.transpose(0, 2, 1))
    sq = sq + jnp.sum(new_xyz ** 2, -1)[:, :, None] + jnp.sum(xyz ** 2, -1)[:, None, :]
    _, idx = jax.lax.top_k(-sq, NS)
    nn3 = new_norm[..., None]
    grouped_xyz = _index_points(xyz, idx)
    local = grouped_xyz - new_xyz[:, :, None, :]
    dist_plane = jnp.matmul(local, nn3)
    proj = local - dist_plane * new_norm[:, :, None, :]
    unit, plen = _safe_unit(proj)
    li = jnp.argmax(plen[..., 0], axis=2)
    vref = jnp.take_along_axis(unit, jnp.broadcast_to(li[:, :, None, None], (b, S, 1, 3)), axis=2)
    dots = jnp.matmul(unit, vref.reshape(b, S, 3, 1))
    sgn = jnp.cross(unit, jnp.broadcast_to(vref, unit.shape))
    sgn = jnp.sign(jnp.matmul(sgn, nn3))
    sgn = sgn.at[:, :, 0, 0].set(1.0)
    dots = sgn * dots - (1.0 - sgn)
    order = jnp.argsort(-dots[..., 0], axis=2)
    dots_sorted = jnp.take_along_axis(dots, order[..., None], axis=2)
    idx_ordered = jnp.take_along_axis(idx, order, axis=2)
    g_xyz = _index_points(xyz, idx_ordered)
    g_local = g_xyz - new_xyz[:, :, None, :]
    g_unit, g_len = _safe_unit(g_local)
    g_norm = _index_points(norm, idx_ordered)
    a0 = jnp.matmul(g_unit, nn3)
    a1 = jnp.sum(g_unit * g_norm, -1, keepdims=True)
    an = jnp.arccos(jnp.clip(jnp.matmul(g_norm, nn3), -1 + eps, 1 - eps))
    an = jnp.where(a0 < a1, 1.0, -1.0) * an
    inner = g_local - jnp.roll(g_local, 1, axis=2)
    iunit, _ = _safe_unit(inner)
    ia0 = jnp.sum(iunit * g_norm, -1, keepdims=True)
    ia1 = jnp.sum(iunit * jnp.roll(g_norm, 1, axis=2), -1, keepdims=True)
    ia2 = jnp.arccos(jnp.clip(jnp.sum(g_norm * jnp.roll(g_norm, 1, axis=2), -1, keepdims=True), -1 + eps, 1 - eps))
    ia2 = jnp.where(ia0 < ia1, 1.0, -1.0) * ia2
    pfeat = dots_sorted - jnp.roll(dots_sorted, 1, axis=2)
    pfeat = pfeat.at[:, :, 0, 0].set(-3.0 - dots_sorted[:, :, -1, 0])
    ri = jnp.concatenate([g_len, pfeat, a0, a1, an, ia0, ia1, ia2], axis=-1)
    h = _bn_relu(jnp.matmul(ri, pW0.T) + pb0, pg0, pbt0)
    h = _bn_relu(jnp.matmul(h, pW1.T) + pb1, pg1, pbt1)
    gp = _index_points(points, idx_ordered)
    h = jnp.concatenate([h, gp], axis=-1)
    h = _bn_relu(jnp.matmul(h, mW0.T) + mb0, mg0, mbt0)
    h = _bn_relu(jnp.matmul(h, mW1.T) + mb1, mg1, mbt1)
    out = _maxpool(h.reshape(B * S, NS, 256)).reshape(B, S, 256)
    return new_xyz, new_norm, out.transpose(0, 2, 1)


# speedup vs baseline: 1.3722x; 1.3722x over previous
import jax
import jax.numpy as jnp
from jax.experimental import pallas as pl

B, N, S, NS, DP = 4, 8192, 1024, 64, 128


def _safe_unit(v):
    l = jnp.linalg.norm(v, axis=-1, keepdims=True)
    d = jnp.where(l > 0, l, 1.0)
    u = jnp.where(l > 0, v / d, 0.0)
    return u, l


def _index_points(points, idx):
    b = points.shape[0]
    flat = idx.reshape(b, -1)
    out = jnp.take_along_axis(points, flat[:, :, None], axis=1)
    return out.reshape(idx.shape + (points.shape[-1],))


def _fps(xyz, npoint):
    b, n, _ = xyz.shape
    def body(i, state):
        idxs, dists, far = state
        idxs = idxs.at[:, i].set(far)
        centroid = jnp.take_along_axis(xyz, far[:, None, None], axis=1)
        d = jnp.sum((xyz - centroid) ** 2, -1)
        dists = jnp.minimum(dists, d)
        far = jnp.argmax(dists, -1).astype(jnp.int32)
        return idxs, dists, far
    idxs = jnp.zeros((b, npoint), dtype=jnp.int32)
    dists = jnp.full((b, n), 1e10, dtype=xyz.dtype)
    far = jnp.zeros((b,), dtype=jnp.int32)
    idxs, _, _ = jax.lax.fori_loop(0, npoint, body, (idxs, dists, far))
    return idxs


def _fps_kernel(xt_ref, nt_ref, nxyz_ref, nnorm_ref):
    x = xt_ref[:, 0]
    y = xt_ref[:, 1]
    z = xt_ref[:, 2]
    nx = nt_ref[:, 0]
    ny = nt_ref[:, 1]
    nz = nt_ref[:, 2]
    r_iota = jax.lax.broadcasted_iota(jnp.int32, (B, 64, 128), 1)
    c_iota = jax.lax.broadcasted_iota(jnp.int32, (B, 64, 128), 2)
    flatidx = r_iota * 128 + c_iota
    BIG = jnp.int32(2 ** 30)

    def body(i, state):
        dists, far = state
        mask = flatidx == far
        maskf = mask.astype(jnp.float32)
        cx = jnp.sum(x * maskf, axis=(1, 2), keepdims=True)
        cy = jnp.sum(y * maskf, axis=(1, 2), keepdims=True)
        cz = jnp.sum(z * maskf, axis=(1, 2), keepdims=True)
        cnx = jnp.sum(nx * maskf, axis=(1, 2), keepdims=True)
        cny = jnp.sum(ny * maskf, axis=(1, 2), keepdims=True)
        cnz = jnp.sum(nz * maskf, axis=(1, 2), keepdims=True)
        nxyz_ref[:, pl.ds(i, 1), :] = jnp.concatenate(
            [cx[:, 0], cy[:, 0], cz[:, 0]], axis=-1)[:, None, :]
        nnorm_ref[:, pl.ds(i, 1), :] = jnp.concatenate(
            [cnx[:, 0], cny[:, 0], cnz[:, 0]], axis=-1)[:, None, :]
        d = (x - cx) ** 2 + (y - cy) ** 2 + (z - cz) ** 2
        dists = jnp.minimum(dists, d)
        m = jnp.max(dists, axis=(1, 2), keepdims=True)
        far = jnp.min(jnp.where(dists == m, flatidx, BIG),
                      axis=(1, 2), keepdims=True)
        return dists, far

    dists0 = jnp.full((B, 64, 128), 1e10, dtype=jnp.float32)
    far0 = jnp.zeros((B, 1, 1), dtype=jnp.int32)
    jax.lax.fori_loop(0, S, body, (dists0, far0))


def _fps_pallas(xyz, norm):
    xt = xyz.transpose(0, 2, 1).reshape(B, 3, 64, 128)
    nt = norm.transpose(0, 2, 1).reshape(B, 3, 64, 128)
    return pl.pallas_call(
        _fps_kernel,
        out_shape=(jax.ShapeDtypeStruct((B, S, 3), jnp.float32),
                   jax.ShapeDtypeStruct((B, S, 3), jnp.float32)),
    )(xt, nt)


def _bn_relu(x, g, b):
    m = jnp.mean(x, axis=(0, 1, 2), keepdims=True)
    v = jnp.var(x, axis=(0, 1, 2), keepdims=True)
    return jax.nn.relu((x - m) / jnp.sqrt(v + 1e-5) * g + b)


def _maxpool_kernel(h_ref, o_ref):
    o_ref[...] = jnp.max(h_ref[...], axis=1)


def _maxpool(h):
    R, K, C = h.shape
    BLK = 128
    return pl.pallas_call(
        _maxpool_kernel,
        grid=(R // BLK,),
        in_specs=[pl.BlockSpec((BLK, K, C), lambda i: (i, 0, 0))],
        out_specs=pl.BlockSpec((BLK, C), lambda i: (i, 0)),
        out_shape=jax.ShapeDtypeStruct((R, C), h.dtype),
    )(h)


def kernel(xyz, norm, points, pW0, pb0, pg0, pbt0, pW1, pb1, pg1, pbt1,
           mW0, mb0, mg0, mbt0, mW1, mb1, mg1, mbt1):
    eps = 1e-7
    b = xyz.shape[0]
    new_xyz, new_norm = _fps_pallas(xyz, norm)
    sq = -2.0 * jnp.matmul(new_xyz, xyz.transpose(0, 2, 1))
    sq = sq + jnp.sum(new_xyz ** 2, -1)[:, :, None] + jnp.sum(xyz ** 2, -1)[:, None, :]
    _, idx = jax.lax.top_k(-sq, NS)
    nn3 = new_norm[..., None]
    grouped_xyz = _index_points(xyz, idx)
    local = grouped_xyz - new_xyz[:, :, None, :]
    dist_plane = jnp.matmul(local, nn3)
    proj = local - dist_plane * new_norm[:, :, None, :]
    unit, plen = _safe_unit(proj)
    li = jnp.argmax(plen[..., 0], axis=2)
    vref = jnp.take_along_axis(unit, jnp.broadcast_to(li[:, :, None, None], (b, S, 1, 3)), axis=2)
    dots = jnp.matmul(unit, vref.reshape(b, S, 3, 1))
    sgn = jnp.cross(unit, jnp.broadcast_to(vref, unit.shape))
    sgn = jnp.sign(jnp.matmul(sgn, nn3))
    sgn = sgn.at[:, :, 0, 0].set(1.0)
    dots = sgn * dots - (1.0 - sgn)
    order = jnp.argsort(-dots[..., 0], axis=2)
    dots_sorted = jnp.take_along_axis(dots, order[..., None], axis=2)
    idx_ordered = jnp.take_along_axis(idx, order, axis=2)
    g_xyz = _index_points(xyz, idx_ordered)
    g_local = g_xyz - new_xyz[:, :, None, :]
    g_unit, g_len = _safe_unit(g_local)
    g_norm = _index_points(norm, idx_ordered)
    a0 = jnp.matmul(g_unit, nn3)
    a1 = jnp.sum(g_unit * g_norm, -1, keepdims=True)
    an = jnp.arccos(jnp.clip(jnp.matmul(g_norm, nn3), -1 + eps, 1 - eps))
    an = jnp.where(a0 < a1, 1.0, -1.0) * an
    inner = g_local - jnp.roll(g_local, 1, axis=2)
    iunit, _ = _safe_unit(inner)
    ia0 = jnp.sum(iunit * g_norm, -1, keepdims=True)
    ia1 = jnp.sum(iunit * jnp.roll(g_norm, 1, axis=2), -1, keepdims=True)
    ia2 = jnp.arccos(jnp.clip(jnp.sum(g_norm * jnp.roll(g_norm, 1, axis=2), -1, keepdims=True), -1 + eps, 1 - eps))
    ia2 = jnp.where(ia0 < ia1, 1.0, -1.0) * ia2
    pfeat = dots_sorted - jnp.roll(dots_sorted, 1, axis=2)
    pfeat = pfeat.at[:, :, 0, 0].set(-3.0 - dots_sorted[:, :, -1, 0])
    ri = jnp.concatenate([g_len, pfeat, a0, a1, an, ia0, ia1, ia2], axis=-1)
    h = _bn_relu(jnp.matmul(ri, pW0.T) + pb0, pg0, pbt0)
    h = _bn_relu(jnp.matmul(h, pW1.T) + pb1, pg1, pbt1)
    gp = _index_points(points, idx_ordered)
    h = jnp.concatenate([h, gp], axis=-1)
    h = _bn_relu(jnp.matmul(h, mW0.T) + mb0, mg0, mbt0)
    h = _bn_relu(jnp.matmul(h, mW1.T) + mb1, mg1, mbt1)
    out = _maxpool(h.reshape(B * S, NS, 256)).reshape(B, S, 256)
    return new_xyz, new_norm, out.transpose(0, 2, 1)


# speedup vs baseline: 3.2621x; 2.3773x over previous
import functools

import jax
import jax.numpy as jnp
from jax import lax
from jax.experimental import pallas as pl
from jax.experimental.pallas import tpu as pltpu, tpu_sc as plsc

B, N, S, NS, DP = 4, 8192, 1024, 64, 128
NW = 32


def _sc_gather(table, idx_flat, chunk=128):
    V, D = table.shape
    M = idx_flat.shape[0]
    per_w = M // NW
    n_ch = per_w // chunk
    mesh = plsc.VectorSubcoreMesh(core_axis_name="c", subcore_axis_name="s",
                                  num_cores=2, num_subcores=16)

    @functools.partial(
        pl.kernel, mesh=mesh,
        out_type=jax.ShapeDtypeStruct((M, D), jnp.float32),
        compiler_params=pltpu.CompilerParams(use_tc_tiling_on_sc=False),
        scratch_types=[
            pltpu.VMEM((per_w,), jnp.int32),
            pltpu.VMEM((chunk, D), jnp.float32),
            pltpu.SemaphoreType.DMA,
        ],
    )
    def k(table_hbm, idx_hbm, out_hbm, idx_v, rows_v, sem):
        wid = lax.axis_index("s") * 2 + lax.axis_index("c")
        base = wid * per_w
        pltpu.sync_copy(idx_hbm.at[pl.ds(base, per_w)], idx_v)

        def body(j, carry):
            pltpu.async_copy(
                table_hbm.at[idx_v.at[pl.ds(j * chunk, chunk)]],
                rows_v, sem).wait()
            pltpu.sync_copy(rows_v, out_hbm.at[pl.ds(base + j * chunk, chunk)])
            return carry

        lax.fori_loop(0, n_ch, body, 0)

    return k(table, idx_flat)


def _safe_unit(v):
    l = jnp.linalg.norm(v, axis=-1, keepdims=True)
    d = jnp.where(l > 0, l, 1.0)
    u = jnp.where(l > 0, v / d, 0.0)
    return u, l


def _index_points(points, idx):
    b = points.shape[0]
    flat = idx.reshape(b, -1)
    out = jnp.take_along_axis(points, flat[:, :, None], axis=1)
    return out.reshape(idx.shape + (points.shape[-1],))


def _fps(xyz, npoint):
    b, n, _ = xyz.shape
    def body(i, state):
        idxs, dists, far = state
        idxs = idxs.at[:, i].set(far)
        centroid = jnp.take_along_axis(xyz, far[:, None, None], axis=1)
        d = jnp.sum((xyz - centroid) ** 2, -1)
        dists = jnp.minimum(dists, d)
        far = jnp.argmax(dists, -1).astype(jnp.int32)
        return idxs, dists, far
    idxs = jnp.zeros((b, npoint), dtype=jnp.int32)
    dists = jnp.full((b, n), 1e10, dtype=xyz.dtype)
    far = jnp.zeros((b,), dtype=jnp.int32)
    idxs, _, _ = jax.lax.fori_loop(0, npoint, body, (idxs, dists, far))
    return idxs


def _fps_kernel(xt_ref, nt_ref, nxyz_ref, nnorm_ref):
    x = xt_ref[:, 0]
    y = xt_ref[:, 1]
    z = xt_ref[:, 2]
    nx = nt_ref[:, 0]
    ny = nt_ref[:, 1]
    nz = nt_ref[:, 2]
    r_iota = jax.lax.broadcasted_iota(jnp.int32, (B, 64, 128), 1)
    c_iota = jax.lax.broadcasted_iota(jnp.int32, (B, 64, 128), 2)
    flatidx = r_iota * 128 + c_iota
    BIG = jnp.int32(2 ** 30)

    def body(i, state):
        dists, far = state
        mask = flatidx == far
        maskf = mask.astype(jnp.float32)
        cx = jnp.sum(x * maskf, axis=(1, 2), keepdims=True)
        cy = jnp.sum(y * maskf, axis=(1, 2), keepdims=True)
        cz = jnp.sum(z * maskf, axis=(1, 2), keepdims=True)
        cnx = jnp.sum(nx * maskf, axis=(1, 2), keepdims=True)
        cny = jnp.sum(ny * maskf, axis=(1, 2), keepdims=True)
        cnz = jnp.sum(nz * maskf, axis=(1, 2), keepdims=True)
        nxyz_ref[:, pl.ds(i, 1), :] = jnp.concatenate(
            [cx[:, 0], cy[:, 0], cz[:, 0]], axis=-1)[:, None, :]
        nnorm_ref[:, pl.ds(i, 1), :] = jnp.concatenate(
            [cnx[:, 0], cny[:, 0], cnz[:, 0]], axis=-1)[:, None, :]
        d = (x - cx) ** 2 + (y - cy) ** 2 + (z - cz) ** 2
        dists = jnp.minimum(dists, d)
        m = jnp.max(dists, axis=(1, 2), keepdims=True)
        far = jnp.min(jnp.where(dists == m, flatidx, BIG),
                      axis=(1, 2), keepdims=True)
        return dists, far

    dists0 = jnp.full((B, 64, 128), 1e10, dtype=jnp.float32)
    far0 = jnp.zeros((B, 1, 1), dtype=jnp.int32)
    jax.lax.fori_loop(0, S, body, (dists0, far0))


def _fps_pallas(xyz, norm):
    xt = xyz.transpose(0, 2, 1).reshape(B, 3, 64, 128)
    nt = norm.transpose(0, 2, 1).reshape(B, 3, 64, 128)
    return pl.pallas_call(
        _fps_kernel,
        out_shape=(jax.ShapeDtypeStruct((B, S, 3), jnp.float32),
                   jax.ShapeDtypeStruct((B, S, 3), jnp.float32)),
    )(xt, nt)


def _bn_relu(x, g, b):
    m = jnp.mean(x, axis=(0, 1, 2), keepdims=True)
    v = jnp.var(x, axis=(0, 1, 2), keepdims=True)
    return jax.nn.relu((x - m) / jnp.sqrt(v + 1e-5) * g + b)


def _maxpool_kernel(h_ref, o_ref):
    o_ref[...] = jnp.max(h_ref[...], axis=1)


def _maxpool(h):
    R, K, C = h.shape
    BLK = 128
    return pl.pallas_call(
        _maxpool_kernel,
        grid=(R // BLK,),
        in_specs=[pl.BlockSpec((BLK, K, C), lambda i: (i, 0, 0))],
        out_specs=pl.BlockSpec((BLK, C), lambda i: (i, 0)),
        out_shape=jax.ShapeDtypeStruct((R, C), h.dtype),
    )(h)


def kernel(xyz, norm, points, pW0, pb0, pg0, pbt0, pW1, pb1, pg1, pbt1,
           mW0, mb0, mg0, mbt0, mW1, mb1, mg1, mbt1):
    eps = 1e-7
    b = xyz.shape[0]
    new_xyz, new_norm = _fps_pallas(xyz, norm)
    sq = -2.0 * jnp.matmul(new_xyz, xyz.transpose(0, 2, 1))
    sq = sq + jnp.sum(new_xyz ** 2, -1)[:, :, None] + jnp.sum(xyz ** 2, -1)[:, None, :]
    _, idx = jax.lax.top_k(-sq, NS)
    nn3 = new_norm[..., None]
    batch_off = (jax.lax.iota(jnp.int32, B) * N)[:, None, None]
    xyzn_table = jnp.concatenate(
        [xyz, norm, jnp.zeros((B, N, 10), jnp.float32)], axis=-1
    ).reshape(B * N, 16)
    grouped_xyzn = _sc_gather(
        xyzn_table, (idx + batch_off).reshape(-1)).reshape(B, S, NS, 16)
    grouped_xyz = grouped_xyzn[..., 0:3]
    local = grouped_xyz - new_xyz[:, :, None, :]
    dist_plane = jnp.matmul(local, nn3)
    proj = local - dist_plane * new_norm[:, :, None, :]
    unit, plen = _safe_unit(proj)
    li = jnp.argmax(plen[..., 0], axis=2)
    vref = jnp.take_along_axis(unit, jnp.broadcast_to(li[:, :, None, None], (b, S, 1, 3)), axis=2)
    dots = jnp.matmul(unit, vref.reshape(b, S, 3, 1))
    sgn = jnp.cross(unit, jnp.broadcast_to(vref, unit.shape))
    sgn = jnp.sign(jnp.matmul(sgn, nn3))
    sgn = sgn.at[:, :, 0, 0].set(1.0)
    dots = sgn * dots - (1.0 - sgn)
    order = jnp.argsort(-dots[..., 0], axis=2)
    dots_sorted = jnp.take_along_axis(dots, order[..., None], axis=2)
    idx_ordered = jnp.take_along_axis(idx, order, axis=2)
    idxo_flat = (idx_ordered + batch_off).reshape(-1)
    g_xyzn = _sc_gather(xyzn_table, idxo_flat).reshape(B, S, NS, 16)
    g_xyz = g_xyzn[..., 0:3]
    g_local = g_xyz - new_xyz[:, :, None, :]
    g_unit, g_len = _safe_unit(g_local)
    g_norm = g_xyzn[..., 3:6]
    a0 = jnp.matmul(g_unit, nn3)
    a1 = jnp.sum(g_unit * g_norm, -1, keepdims=True)
    an = jnp.arccos(jnp.clip(jnp.matmul(g_norm, nn3), -1 + eps, 1 - eps))
    an = jnp.where(a0 < a1, 1.0, -1.0) * an
    inner = g_local - jnp.roll(g_local, 1, axis=2)
    iunit, _ = _safe_unit(inner)
    ia0 = jnp.sum(iunit * g_norm, -1, keepdims=True)
    ia1 = jnp.sum(iunit * jnp.roll(g_norm, 1, axis=2), -1, keepdims=True)
    ia2 = jnp.arccos(jnp.clip(jnp.sum(g_norm * jnp.roll(g_norm, 1, axis=2), -1, keepdims=True), -1 + eps, 1 - eps))
    ia2 = jnp.where(ia0 < ia1, 1.0, -1.0) * ia2
    pfeat = dots_sorted - jnp.roll(dots_sorted, 1, axis=2)
    pfeat = pfeat.at[:, :, 0, 0].set(-3.0 - dots_sorted[:, :, -1, 0])
    ri = jnp.concatenate([g_len, pfeat, a0, a1, an, ia0, ia1, ia2], axis=-1)
    h = _bn_relu(jnp.matmul(ri, pW0.T) + pb0, pg0, pbt0)
    h = _bn_relu(jnp.matmul(h, pW1.T) + pb1, pg1, pbt1)
    gp = _sc_gather(points.reshape(B * N, DP), idxo_flat).reshape(B, S, NS, DP)
    h = jnp.concatenate([h, gp], axis=-1)
    h = _bn_relu(jnp.matmul(h, mW0.T) + mb0, mg0, mbt0)
    h = _bn_relu(jnp.matmul(h, mW1.T) + mb1, mg1, mbt1)
    out = _maxpool(h.reshape(B * S, NS, 256)).reshape(B, S, 256)
    return new_xyz, new_norm, out.transpose(0, 2, 1)


# speedup vs baseline: 3.8474x; 1.1794x over previous
import functools

import jax
import jax.numpy as jnp
from jax import lax
from jax.experimental import pallas as pl
from jax.experimental.pallas import tpu as pltpu, tpu_sc as plsc

B, N, S, NS, DP = 4, 8192, 1024, 64, 128
NW = 32


def _sc_gather(table, idx_flat, chunk=128):
    V, D = table.shape
    M = idx_flat.shape[0]
    per_w = M // NW
    n_ch = per_w // chunk
    mesh = plsc.VectorSubcoreMesh(core_axis_name="c", subcore_axis_name="s",
                                  num_cores=2, num_subcores=16)

    @functools.partial(
        pl.kernel, mesh=mesh,
        out_type=jax.ShapeDtypeStruct((M, D), jnp.float32),
        compiler_params=pltpu.CompilerParams(use_tc_tiling_on_sc=False),
        scratch_types=[
            pltpu.VMEM((per_w,), jnp.int32),
            pltpu.VMEM((chunk, D), jnp.float32),
            pltpu.SemaphoreType.DMA,
        ],
    )
    def k(table_hbm, idx_hbm, out_hbm, idx_v, rows_v, sem):
        wid = lax.axis_index("s") * 2 + lax.axis_index("c")
        base = wid * per_w
        pltpu.sync_copy(idx_hbm.at[pl.ds(base, per_w)], idx_v)

        def body(j, carry):
            pltpu.async_copy(
                table_hbm.at[idx_v.at[pl.ds(j * chunk, chunk)]],
                rows_v, sem).wait()
            pltpu.sync_copy(rows_v, out_hbm.at[pl.ds(base + j * chunk, chunk)])
            return carry

        lax.fori_loop(0, n_ch, body, 0)

    return k(table, idx_flat)


def _safe_unit(v):
    l = jnp.linalg.norm(v, axis=-1, keepdims=True)
    d = jnp.where(l > 0, l, 1.0)
    u = jnp.where(l > 0, v / d, 0.0)
    return u, l


def _index_points(points, idx):
    b = points.shape[0]
    flat = idx.reshape(b, -1)
    out = jnp.take_along_axis(points, flat[:, :, None], axis=1)
    return out.reshape(idx.shape + (points.shape[-1],))


def _fps(xyz, npoint):
    b, n, _ = xyz.shape
    def body(i, state):
        idxs, dists, far = state
        idxs = idxs.at[:, i].set(far)
        centroid = jnp.take_along_axis(xyz, far[:, None, None], axis=1)
        d = jnp.sum((xyz - centroid) ** 2, -1)
        dists = jnp.minimum(dists, d)
        far = jnp.argmax(dists, -1).astype(jnp.int32)
        return idxs, dists, far
    idxs = jnp.zeros((b, npoint), dtype=jnp.int32)
    dists = jnp.full((b, n), 1e10, dtype=xyz.dtype)
    far = jnp.zeros((b,), dtype=jnp.int32)
    idxs, _, _ = jax.lax.fori_loop(0, npoint, body, (idxs, dists, far))
    return idxs


def _fps_kernel(xt_ref, nt_ref, nxyz_ref, nnorm_ref):
    x = xt_ref[:, 0]
    y = xt_ref[:, 1]
    z = xt_ref[:, 2]
    nx = nt_ref[:, 0]
    ny = nt_ref[:, 1]
    nz = nt_ref[:, 2]
    r_iota = jax.lax.broadcasted_iota(jnp.int32, (B, 64, 128), 1)
    c_iota = jax.lax.broadcasted_iota(jnp.int32, (B, 64, 128), 2)
    flatidx = r_iota * 128 + c_iota
    BIG = jnp.int32(2 ** 30)

    def body(i, state):
        dists, far = state
        mask = flatidx == far
        maskf = mask.astype(jnp.float32)
        cx = jnp.sum(x * maskf, axis=(1, 2), keepdims=True)
        cy = jnp.sum(y * maskf, axis=(1, 2), keepdims=True)
        cz = jnp.sum(z * maskf, axis=(1, 2), keepdims=True)
        cnx = jnp.sum(nx * maskf, axis=(1, 2), keepdims=True)
        cny = jnp.sum(ny * maskf, axis=(1, 2), keepdims=True)
        cnz = jnp.sum(nz * maskf, axis=(1, 2), keepdims=True)
        nxyz_ref[:, pl.ds(i, 1), :] = jnp.concatenate(
            [cx[:, 0], cy[:, 0], cz[:, 0]], axis=-1)[:, None, :]
        nnorm_ref[:, pl.ds(i, 1), :] = jnp.concatenate(
            [cnx[:, 0], cny[:, 0], cnz[:, 0]], axis=-1)[:, None, :]
        d = (x - cx) ** 2 + (y - cy) ** 2 + (z - cz) ** 2
        dists = jnp.minimum(dists, d)
        m = jnp.max(dists, axis=(1, 2), keepdims=True)
        far = jnp.min(jnp.where(dists == m, flatidx, BIG),
                      axis=(1, 2), keepdims=True)
        return dists, far

    dists0 = jnp.full((B, 64, 128), 1e10, dtype=jnp.float32)
    far0 = jnp.zeros((B, 1, 1), dtype=jnp.int32)
    jax.lax.fori_loop(0, S, body, (dists0, far0))


def _fps_pallas(xyz, norm):
    xt = xyz.transpose(0, 2, 1).reshape(B, 3, 64, 128)
    nt = norm.transpose(0, 2, 1).reshape(B, 3, 64, 128)
    return pl.pallas_call(
        _fps_kernel,
        out_shape=(jax.ShapeDtypeStruct((B, S, 3), jnp.float32),
                   jax.ShapeDtypeStruct((B, S, 3), jnp.float32)),
    )(xt, nt)


QB = 32


def _ce(key, pay, j, up):
    i = jax.lax.broadcasted_iota(jnp.int32, (1, key.shape[1], 1), 1)
    mask_hi = (i & j) != 0
    pkey = jnp.where(mask_hi, jnp.roll(key, j, axis=1),
                     jnp.roll(key, -j, axis=1))
    ppay = jnp.where(mask_hi, jnp.roll(pay, j, axis=1),
                     jnp.roll(pay, -j, axis=1))
    keep_min = jnp.logical_xor(mask_hi, up)
    take = (keep_min & (pkey < key)) | (~keep_min & (pkey > key))
    return jnp.where(take, pkey, key), jnp.where(take, ppay, pay)


def _knn_kernel(q_ref, pk_ref, idx_ref):
    q = q_ref[0]
    pk = pk_ref[0]
    mm = jnp.dot(q.astype(jnp.bfloat16), pk.astype(jnp.bfloat16),
                 preferred_element_type=jnp.float32)
    q2 = q[:, 0:1] ** 2 + q[:, 1:2] ** 2 + q[:, 2:3] ** 2
    p2 = pk[0:1] ** 2 + pk[1:2] ** 2 + pk[2:3] ** 2
    sq = (-2.0 * mm) + q2 + p2
    key = sq.reshape(QB, 64, 128)
    pay = (jax.lax.broadcasted_iota(jnp.int32, (QB, 64, 128), 1) * 128
           + jax.lax.broadcasted_iota(jnp.int32, (QB, 64, 128), 2))
    i = jax.lax.broadcasted_iota(jnp.int32, (1, 64, 1), 1)
    for k in (2, 4, 8, 16, 32, 64):
        up = (i & k) == 0
        j = k >> 1
        while j >= 1:
            key, pay = _ce(key, pay, j, up)
            j >>= 1
    def rev64(x):
        for j in (32, 16, 8, 4, 2, 1):
            x = jnp.where((i & j) != 0, jnp.roll(x, j, axis=1),
                          jnp.roll(x, -j, axis=1))
        return x

    for _ in range(7):
        w = key.shape[2] // 2
        ka, kb = key[:, :, :w], rev64(key[:, :, w:])
        pa, pb = pay[:, :, :w], rev64(pay[:, :, w:])
        cond = kb < ka
        key = jnp.where(cond, kb, ka)
        pay = jnp.where(cond, pb, pa)
        for j in (32, 16, 8, 4, 2, 1):
            key, pay = _ce(key, pay, j, True)
    idx_ref[0] = pay[:, :, 0]


def _knn_pallas(new_xyz, xyz):
    qpad = jnp.concatenate(
        [new_xyz, jnp.zeros((B, S, 5), jnp.float32)], axis=-1)
    ppad = jnp.concatenate(
        [xyz.transpose(0, 2, 1), jnp.zeros((B, 5, N), jnp.float32)], axis=1)
    return pl.pallas_call(
        _knn_kernel,
        grid=(B, S // QB),
        in_specs=[pl.BlockSpec((1, QB, 8), lambda b, s: (b, s, 0)),
                  pl.BlockSpec((1, 8, N), lambda b, s: (b, 0, 0))],
        out_specs=pl.BlockSpec((1, QB, NS), lambda b, s: (b, s, 0)),
        out_shape=jax.ShapeDtypeStruct((B, S, NS), jnp.int32),
    )(qpad, ppad)


def _bn_relu(x, g, b):
    m = jnp.mean(x, axis=(0, 1, 2), keepdims=True)
    v = jnp.var(x, axis=(0, 1, 2), keepdims=True)
    return jax.nn.relu((x - m) / jnp.sqrt(v + 1e-5) * g + b)


def _maxpool_kernel(h_ref, o_ref):
    o_ref[...] = jnp.max(h_ref[...], axis=1)


def _maxpool(h):
    R, K, C = h.shape
    BLK = 128
    return pl.pallas_call(
        _maxpool_kernel,
        grid=(R // BLK,),
        in_specs=[pl.BlockSpec((BLK, K, C), lambda i: (i, 0, 0))],
        out_specs=pl.BlockSpec((BLK, C), lambda i: (i, 0)),
        out_shape=jax.ShapeDtypeStruct((R, C), h.dtype),
    )(h)


def kernel(xyz, norm, points, pW0, pb0, pg0, pbt0, pW1, pb1, pg1, pbt1,
           mW0, mb0, mg0, mbt0, mW1, mb1, mg1, mbt1):
    eps = 1e-7
    b = xyz.shape[0]
    new_xyz, new_norm = _fps_pallas(xyz, norm)
    idx = _knn_pallas(new_xyz, xyz)
    nn3 = new_norm[..., None]
    batch_off = (jax.lax.iota(jnp.int32, B) * N)[:, None, None]
    xyzn_table = jnp.concatenate(
        [xyz, norm, jnp.zeros((B, N, 10), jnp.float32)], axis=-1
    ).reshape(B * N, 16)
    grouped_xyzn = _sc_gather(
        xyzn_table, (idx + batch_off).reshape(-1)).reshape(B, S, NS, 16)
    grouped_xyz = grouped_xyzn[..., 0:3]
    local = grouped_xyz - new_xyz[:, :, None, :]
    dist_plane = jnp.matmul(local, nn3)
    proj = local - dist_plane * new_norm[:, :, None, :]
    unit, plen = _safe_unit(proj)
    li = jnp.argmax(plen[..., 0], axis=2)
    vref = jnp.take_along_axis(unit, jnp.broadcast_to(li[:, :, None, None], (b, S, 1, 3)), axis=2)
    dots = jnp.matmul(unit, vref.reshape(b, S, 3, 1))
    sgn = jnp.cross(unit, jnp.broadcast_to(vref, unit.shape))
    sgn = jnp.sign(jnp.matmul(sgn, nn3))
    sgn = sgn.at[:, :, 0, 0].set(1.0)
    dots = sgn * dots - (1.0 - sgn)
    order = jnp.argsort(-dots[..., 0], axis=2)
    dots_sorted = jnp.take_along_axis(dots, order[..., None], axis=2)
    idx_ordered = jnp.take_along_axis(idx, order, axis=2)
    idxo_flat = (idx_ordered + batch_off).reshape(-1)
    g_xyzn = _sc_gather(xyzn_table, idxo_flat).reshape(B, S, NS, 16)
    g_xyz = g_xyzn[..., 0:3]
    g_local = g_xyz - new_xyz[:, :, None, :]
    g_unit, g_len = _safe_unit(g_local)
    g_norm = g_xyzn[..., 3:6]
    a0 = jnp.matmul(g_unit, nn3)
    a1 = jnp.sum(g_unit * g_norm, -1, keepdims=True)
    an = jnp.arccos(jnp.clip(jnp.matmul(g_norm, nn3), -1 + eps, 1 - eps))
    an = jnp.where(a0 < a1, 1.0, -1.0) * an
    inner = g_local - jnp.roll(g_local, 1, axis=2)
    iunit, _ = _safe_unit(inner)
    ia0 = jnp.sum(iunit * g_norm, -1, keepdims=True)
    ia1 = jnp.sum(iunit * jnp.roll(g_norm, 1, axis=2), -1, keepdims=True)
    ia2 = jnp.arccos(jnp.clip(jnp.sum(g_norm * jnp.roll(g_norm, 1, axis=2), -1, keepdims=True), -1 + eps, 1 - eps))
    ia2 = jnp.where(ia0 < ia1, 1.0, -1.0) * ia2
    pfeat = dots_sorted - jnp.roll(dots_sorted, 1, axis=2)
    pfeat = pfeat.at[:, :, 0, 0].set(-3.0 - dots_sorted[:, :, -1, 0])
    ri = jnp.concatenate([g_len, pfeat, a0, a1, an, ia0, ia1, ia2], axis=-1)
    h = _bn_relu(jnp.matmul(ri, pW0.T) + pb0, pg0, pbt0)
    h = _bn_relu(jnp.matmul(h, pW1.T) + pb1, pg1, pbt1)
    gp = _sc_gather(points.reshape(B * N, DP), idxo_flat).reshape(B, S, NS, DP)
    h = jnp.concatenate([h, gp], axis=-1)
    h = _bn_relu(jnp.matmul(h, mW0.T) + mb0, mg0, mbt0)
    h = _bn_relu(jnp.matmul(h, mW1.T) + mb1, mg1, mbt1)
    out = _maxpool(h.reshape(B * S, NS, 256)).reshape(B, S, 256)
    return new_xyz, new_norm, out.transpose(0, 2, 1)
